# baseline (speedup 1.0000x reference)
import functools
import jax
import jax.numpy as jnp
from jax import lax
from jax.experimental import pallas as pl
from jax.experimental.pallas import tpu as pltpu
from jax.experimental.pallas import tpu_sc as plsc

_INF = 1024.0
_TPAD = 136
_CPAD = 256


def _bfs_closed(adj, occ, T):
    Af = jnp.where(adj, 1.0, 0.0).astype(jnp.bfloat16)
    Bf = jnp.where(adj & (occ > 0.5), 1.0, 0.0).astype(jnp.bfloat16)
    dist0 = jnp.where(adj, 1.0, _INF)

    def cond(c):
        d, changed, _, _ = c
        return (d < T) & (changed > 0.5)

    def body(c):
        d, _, F, dist = c
        G = lax.dot_general(Bf, F, (((1,), (0,)), ((), ())),
                            preferred_element_type=jnp.float32)
        Fn = jnp.minimum(jnp.maximum(F, G.astype(jnp.bfloat16)),
                         jnp.bfloat16(1.0))
        isinf = jnp.floor(dist * (1.0 / _INF))
        newlyf = isinf * Fn.astype(jnp.float32)
        changed = jnp.max(newlyf)
        dist = dist + newlyf * ((d + 1).astype(jnp.float32) - _INF)
        return d + 1, changed, Fn, dist

    _, _, _, dist = lax.while_loop(
        cond, body, (jnp.int32(1), jnp.float32(1.0), Af, dist0))
    return dist


def _tc_body(mono_col_ref, tb_ref, a2t_ref, dext_ref, ridx_ref, cidx_ref):
    N = a2t_ref.shape[1]
    T = a2t_ref.shape[2]
    b = pl.program_id(0)
    x = a2t_ref[0]
    tb = tb_ref[0]
    mono_col = mono_col_ref[0]

    lane = lax.broadcasted_iota(jnp.int32, (N, T), 1)
    m = jnp.max(x, axis=1, keepdims=True)
    idx = jnp.min(jnp.where(x == m, lane, T), axis=1, keepdims=True)
    P = (lane == idx).astype(jnp.bfloat16)

    gly_col = (mono_col != -1)
    Pg = P * gly_col.astype(jnp.bfloat16)
    occ = jnp.max(Pg.astype(jnp.float32), axis=0, keepdims=True)

    D = _bfs_closed(tb > 0.0, occ, T)
    Dint = jnp.where(D > 500.0, -1, D.astype(jnp.int32))

    dext_ref[0] = jnp.full((_TPAD, _CPAD), -1, jnp.int32)
    dext_ref[0, 0:T, 0:T] = Dint

    gidx = jnp.where(gly_col, idx, T)
    ridx_ref[0] = b * _TPAD + gidx
    cidx_ref[0] = gidx


def _make_sc_expand(n_rows, n_cols, rows_per_w, tpad_total):
    mesh = plsc.VectorSubcoreMesh(core_axis_name="c", subcore_axis_name="s")
    info = plsc.get_sparse_core_info()
    nc = info.num_cores
    chunks = n_cols // 16

    @functools.partial(
        pl.kernel, mesh=mesh,
        compiler_params=pltpu.CompilerParams(needs_layout_passes=False),
        out_type=jax.ShapeDtypeStruct((n_rows * n_cols,), jnp.int32),
        scratch_types=[
            pltpu.VMEM((rows_per_w,), jnp.int32),
            pltpu.VMEM((rows_per_w, _CPAD), jnp.int32),
            pltpu.VMEM((rows_per_w * _CPAD,), jnp.int32),
            pltpu.VMEM((n_cols,), jnp.int32),
            pltpu.VMEM((rows_per_w * n_cols,), jnp.int32),
            pltpu.SemaphoreType.DMA,
        ],
    )
    def sc_expand(dext_hbm, ridx_hbm, cidx_hbm, out_hbm,
                  ridx_v, rows_v, rowsflat_v, cidx_v, outblk_v, sem):
        wid = lax.axis_index("s") * nc + lax.axis_index("c")
        base = wid * rows_per_w
        batch = base // 512
        pltpu.sync_copy(ridx_hbm.at[pl.ds(base, rows_per_w)], ridx_v)
        pltpu.sync_copy(cidx_hbm.at[pl.ds(batch * 512, n_cols)], cidx_v)
        pltpu.async_copy(dext_hbm.at[ridx_v], rows_v, sem).wait()

        zeros16 = jnp.zeros((16,), jnp.int32)
        lane0 = lax.iota(jnp.int32, 16) == 0

        def row_body(r, carry):
            rvec = zeros16 + r
            for c in range(chunks):
                idxv = cidx_v[pl.ds(c * 16, 16)]
                vals = plsc.load_gather(rows_v, [rvec, idxv])
                outblk_v[pl.ds(r * n_cols + c * 16, 16)] = vals
            dpos = zeros16 + (r * n_cols + (base + r) % 512)
            plsc.store_scatter(outblk_v, [dpos], zeros16, mask=lane0)
            return carry

        lax.fori_loop(0, rows_per_w, row_body, 0)
        pltpu.sync_copy(outblk_v, out_hbm.at[pl.ds(base * n_cols,
                                                   rows_per_w * n_cols)])

    return sc_expand


def kernel(atom_pad_mask, atom_mono_idx, token_bonds, atom_to_token):
    B, N = atom_pad_mask.shape
    T = token_bonds.shape[1]
    tb = jnp.squeeze(token_bonds, -1)
    mono_col = atom_mono_idx.reshape(B, N, 1)

    dext, ridx, cidx = pl.pallas_call(
        _tc_body,
        grid=(B,),
        in_specs=[
            pl.BlockSpec((1, N, 1), lambda b: (b, 0, 0)),
            pl.BlockSpec((1, T, T), lambda b: (b, 0, 0)),
            pl.BlockSpec((1, N, T), lambda b: (b, 0, 0)),
        ],
        out_specs=[
            pl.BlockSpec((1, _TPAD, _CPAD), lambda b: (b, 0, 0)),
            pl.BlockSpec((1, N, 1), lambda b: (b, 0, 0)),
            pl.BlockSpec((1, N, 1), lambda b: (b, 0, 0)),
        ],
        out_shape=[
            jax.ShapeDtypeStruct((B, _TPAD, _CPAD), jnp.int32),
            jax.ShapeDtypeStruct((B, N, 1), jnp.int32),
            jax.ShapeDtypeStruct((B, N, 1), jnp.int32),
        ],
    )(mono_col, tb, atom_to_token)

    n_rows = B * N
    rows_per_w = n_rows // 32
    sc_expand = _make_sc_expand(n_rows, N, rows_per_w, B * _TPAD)
    out = sc_expand(dext.reshape(B * _TPAD, _CPAD),
                    ridx.reshape(n_rows),
                    cidx.reshape(n_rows))
    return out.reshape(B, N, N)


# speedup vs baseline: 1.3253x; 1.3253x over previous
import functools
import jax
import jax.numpy as jnp
from jax import lax
from jax.experimental import pallas as pl
from jax.experimental.pallas import tpu as pltpu
from jax.experimental.pallas import tpu_sc as plsc

_INF = 1024.0
_TPAD = 136
_CPAD = 256


def _bfs_closed(adj, occ, T):
    Af = jnp.where(adj, 1.0, 0.0).astype(jnp.bfloat16)
    Bf = jnp.where(adj & (occ > 0.5), 1.0, 0.0).astype(jnp.bfloat16)
    dist0 = jnp.where(adj, 1.0, _INF)

    def cond(c):
        d, changed, _, _ = c
        return (d < T) & (changed > 0.5)

    def body(c):
        d, _, F, dist = c
        G = lax.dot_general(Bf, F, (((1,), (0,)), ((), ())),
                            preferred_element_type=jnp.float32)
        Fn = jnp.minimum(jnp.maximum(F, G.astype(jnp.bfloat16)),
                         jnp.bfloat16(1.0))
        isinf = jnp.floor(dist * (1.0 / _INF))
        newlyf = isinf * Fn.astype(jnp.float32)
        changed = jnp.max(newlyf)
        dist = dist + newlyf * ((d + 1).astype(jnp.float32) - _INF)
        return d + 1, changed, Fn, dist

    _, _, _, dist = lax.while_loop(
        cond, body, (jnp.int32(1), jnp.float32(1.0), Af, dist0))
    return dist


def _tc_body(mono_col_ref, tb_ref, a2t_ref, dext_ref, ridx_ref, cidx_ref):
    N = a2t_ref.shape[1]
    T = a2t_ref.shape[2]
    b = pl.program_id(0)
    x = a2t_ref[0]
    tb = tb_ref[0]
    mono_col = mono_col_ref[0]

    lane = lax.broadcasted_iota(jnp.int32, (N, T), 1)
    m = jnp.max(x, axis=1, keepdims=True)
    idx = jnp.min(jnp.where(x == m, lane, T), axis=1, keepdims=True)
    P = (lane == idx).astype(jnp.bfloat16)

    gly_col = (mono_col != -1)
    Pg = P * gly_col.astype(jnp.bfloat16)
    occ = jnp.max(Pg.astype(jnp.float32), axis=0, keepdims=True)

    D = _bfs_closed(tb > 0.0, occ, T)
    Dint = jnp.where(D > 500.0, -1, D.astype(jnp.int32))

    dext_ref[0] = jnp.full((_TPAD, _CPAD), -1, jnp.int32)
    dext_ref[0, 0:T, 0:T] = Dint

    gidx = jnp.where(gly_col, idx, T)
    ridx_ref[0] = b * _TPAD + gidx
    cidx_ref[0] = gidx


def _make_sc_expand(n_rows, n_cols, rows_per_w, tpad_total):
    mesh = plsc.VectorSubcoreMesh(core_axis_name="c", subcore_axis_name="s")
    info = plsc.get_sparse_core_info()
    nc = info.num_cores
    chunks = n_cols // 16

    @functools.partial(
        pl.kernel, mesh=mesh,
        compiler_params=pltpu.CompilerParams(needs_layout_passes=False),
        out_type=jax.ShapeDtypeStruct((n_rows * n_cols,), jnp.int32),
        scratch_types=[
            pltpu.VMEM((rows_per_w,), jnp.int32),
            pltpu.VMEM((rows_per_w, _CPAD), jnp.int32),
            pltpu.VMEM((rows_per_w * _CPAD,), jnp.int32),
            pltpu.VMEM((n_cols,), jnp.int32),
            pltpu.VMEM((rows_per_w * n_cols,), jnp.int32),
            pltpu.SemaphoreType.DMA,
        ],
    )
    def sc_expand(dext_hbm, ridx_hbm, cidx_hbm, out_hbm,
                  ridx_v, rows_v, rowsflat_v, cidx_v, outblk_v, sem):
        wid = lax.axis_index("s") * nc + lax.axis_index("c")
        base = wid * rows_per_w
        batch = base // 512
        pltpu.sync_copy(ridx_hbm.at[pl.ds(base, rows_per_w)], ridx_v)
        pltpu.sync_copy(cidx_hbm.at[pl.ds(batch * 512, n_cols)], cidx_v)
        pltpu.async_copy(dext_hbm.at[ridx_v], rows_v, sem).wait()

        zeros16 = jnp.zeros((16,), jnp.int32)
        lane0 = lax.iota(jnp.int32, 16) == 0

        @plsc.parallel_loop(0, rows_per_w * chunks, unroll=4)
        def _(it):
            r = it // chunks
            c = it % chunks
            rvec = zeros16 + r
            idxv = cidx_v[pl.ds(c * 16, 16)]
            vals = plsc.load_gather(rows_v, [rvec, idxv])
            outblk_v[pl.ds(it * 16, 16)] = vals

        @plsc.parallel_loop(0, rows_per_w, unroll=2)
        def _(r):
            rvec = zeros16 + r
            dpos = zeros16 + (r * n_cols + (base + r) % 512)
            plsc.store_scatter(outblk_v, [dpos], zeros16, mask=lane0)
        pltpu.sync_copy(outblk_v, out_hbm.at[pl.ds(base * n_cols,
                                                   rows_per_w * n_cols)])

    return sc_expand


def kernel(atom_pad_mask, atom_mono_idx, token_bonds, atom_to_token):
    B, N = atom_pad_mask.shape
    T = token_bonds.shape[1]
    tb = jnp.squeeze(token_bonds, -1)
    mono_col = atom_mono_idx.reshape(B, N, 1)

    dext, ridx, cidx = pl.pallas_call(
        _tc_body,
        grid=(B,),
        in_specs=[
            pl.BlockSpec((1, N, 1), lambda b: (b, 0, 0)),
            pl.BlockSpec((1, T, T), lambda b: (b, 0, 0)),
            pl.BlockSpec((1, N, T), lambda b: (b, 0, 0)),
        ],
        out_specs=[
            pl.BlockSpec((1, _TPAD, _CPAD), lambda b: (b, 0, 0)),
            pl.BlockSpec((1, N, 1), lambda b: (b, 0, 0)),
            pl.BlockSpec((1, N, 1), lambda b: (b, 0, 0)),
        ],
        out_shape=[
            jax.ShapeDtypeStruct((B, _TPAD, _CPAD), jnp.int32),
            jax.ShapeDtypeStruct((B, N, 1), jnp.int32),
            jax.ShapeDtypeStruct((B, N, 1), jnp.int32),
        ],
    )(mono_col, tb, atom_to_token)

    n_rows = B * N
    rows_per_w = n_rows // 32
    sc_expand = _make_sc_expand(n_rows, N, rows_per_w, B * _TPAD)
    out = sc_expand(dext.reshape(B * _TPAD, _CPAD),
                    ridx.reshape(n_rows),
                    cidx.reshape(n_rows))
    return out.reshape(B, N, N)


# speedup vs baseline: 5.5927x; 4.2199x over previous
import jax
import jax.numpy as jnp
from jax import lax
from jax.experimental import pallas as pl

_INF = 1024.0
_BK = 8


def _bfs_closed_pair(adjs, occs, T):
    nb = len(adjs)
    Bs = jnp.concatenate(
        [jnp.concatenate(
            [jnp.where(adjs[b] & (occs[b] > 0.5), 1.0, 0.0)
             if bb == b else jnp.zeros((T, T))
             for bb in range(nb)], axis=1)
         for b in range(nb)], axis=0).astype(jnp.bfloat16)
    adj = jnp.concatenate(adjs, axis=0)
    Af = jnp.where(adj, 1.0, 0.0).astype(jnp.bfloat16)
    dist0 = jnp.where(adj, 1.0, _INF)

    def cond(c):
        d, changed, _, _ = c
        return (d < T) & (changed > 0.5)

    def body(c):
        d, _, F, dist = c
        G = lax.dot_general(Bs, F, (((1,), (0,)), ((), ())),
                            preferred_element_type=jnp.float32)
        Fn = jnp.minimum(jnp.maximum(F, G.astype(jnp.bfloat16)),
                         jnp.bfloat16(1.0))
        isinf = jnp.floor(dist * (1.0 / _INF))
        newlyf = isinf * Fn.astype(jnp.float32)
        changed = jnp.max(newlyf)
        dist = dist + newlyf * ((d + 1).astype(jnp.float32) - _INF)
        return d + 1, changed, Fn, dist

    _, _, _, dist = lax.while_loop(
        cond, body, (jnp.int32(1), jnp.float32(1.0), Af, dist0))
    return [lax.slice(dist, (b * T, 0), ((b + 1) * T, T)) for b in range(nb)]


def _batch_assign(x, mono_col, N, T):
    lane = lax.broadcasted_iota(jnp.int32, (N, T), 1)
    m = jnp.max(x, axis=1, keepdims=True)
    idx = jnp.min(jnp.where(x == m, lane, T), axis=1, keepdims=True)
    P = (lane == idx).astype(jnp.bfloat16)

    gly_col = (mono_col != -1)
    Pg = P * gly_col.astype(jnp.bfloat16)
    occ = jnp.max(Pg.astype(jnp.float32), axis=0, keepdims=True)
    return P, gly_col, occ


def _expand(P, D, gly_col, gly_row, N):
    R = lax.dot_general(P, D.astype(jnp.bfloat16), (((1,), (0,)), ((), ())),
                        preferred_element_type=jnp.float32)
    O = lax.dot_general(R.astype(jnp.bfloat16), P, (((1,), (1,)), ((), ())),
                        preferred_element_type=jnp.float32)
    li2 = lax.broadcasted_iota(jnp.int32, (N, N), 1)
    si2 = lax.broadcasted_iota(jnp.int32, (N, N), 0)
    vals = O.astype(jnp.int32)
    vals = jnp.where(O > 500.0, -1, vals)
    vals = jnp.where(gly_col & gly_row, vals, -1)
    vals = jnp.where(li2 == si2, 0, vals)
    return vals


def _fw_body(mono_col_ref, mono_row_ref, tb_ref, a2t_ref, out_ref):
    B = a2t_ref.shape[0]
    N = a2t_ref.shape[1]
    T = a2t_ref.shape[2]
    assigns = [_batch_assign(a2t_ref[b], mono_col_ref[b], N, T)
               for b in range(B)]
    Ds = _bfs_closed_pair([tb_ref[b] > 0.0 for b in range(B)],
                          [a[2] for a in assigns], T)
    for b in range(B):
        P, gly_col, _ = assigns[b]
        out_ref[b] = _expand(P, Ds[b], gly_col, mono_row_ref[b] != -1, N)


def kernel(atom_pad_mask, atom_mono_idx, token_bonds, atom_to_token):
    B, N = atom_pad_mask.shape
    T = token_bonds.shape[1]
    tb = jnp.squeeze(token_bonds, -1)
    mono_col = atom_mono_idx.reshape(B, N, 1)
    mono_row = atom_mono_idx.reshape(B, 1, N)
    out = pl.pallas_call(
        _fw_body,
        out_shape=jax.ShapeDtypeStruct((B, N, N), jnp.int32),
    )(mono_col, mono_row, tb, atom_to_token)
    return out


# speedup vs baseline: 5.6003x; 1.0014x over previous
import jax
import jax.numpy as jnp
from jax import lax
from jax.experimental import pallas as pl

_INF = 1024.0


def _bfs_closed_pair(adjs, occs, T):
    nb = len(adjs)
    Bs = jnp.concatenate(
        [jnp.concatenate(
            [jnp.where(adjs[b] & (occs[b] > 0.5), 1.0, 0.0)
             if bb == b else jnp.zeros((T, T))
             for bb in range(nb)], axis=1)
         for b in range(nb)], axis=0).astype(jnp.bfloat16)
    adj = jnp.concatenate(adjs, axis=0)
    Af = jnp.where(adj, 1.0, 0.0).astype(jnp.bfloat16)
    dist0 = jnp.where(adj, 1.0, _INF)

    def cond(c):
        d, changed, _, _ = c
        return (d < T) & (changed > 0.5)

    def body(c):
        d, _, F, dist = c
        G = lax.dot_general(Bs, F, (((1,), (0,)), ((), ())),
                            preferred_element_type=jnp.float32)
        Fn = jnp.minimum(jnp.maximum(F, G.astype(jnp.bfloat16)),
                         jnp.bfloat16(1.0))
        isinf = jnp.floor(dist * (1.0 / _INF))
        newlyf = isinf * Fn.astype(jnp.float32)
        changed = jnp.max(newlyf)
        dist = dist + newlyf * ((d + 1).astype(jnp.float32) - _INF)
        return d + 1, changed, Fn, dist

    _, _, _, dist = lax.while_loop(
        cond, body, (jnp.int32(1), jnp.float32(1.0), Af, dist0))
    return [lax.slice(dist, (b * T, 0), ((b + 1) * T, T)) for b in range(nb)]


def _batch_assign(x, mono_col, N, T):
    lane = lax.broadcasted_iota(jnp.int32, (N, T), 1)
    m = jnp.max(x, axis=1, keepdims=True)
    idx = jnp.min(jnp.where(x == m, lane, T), axis=1, keepdims=True)
    P = (lane == idx).astype(jnp.bfloat16)

    gly_col = (mono_col != -1)
    Pg = P * gly_col.astype(jnp.bfloat16)
    occ = jnp.max(Pg.astype(jnp.float32), axis=0, keepdims=True)
    return P, gly_col, occ


def _expand(P, D, gly_col, gly_row, N):
    R = lax.dot_general(P, D.astype(jnp.bfloat16), (((1,), (0,)), ((), ())),
                        preferred_element_type=jnp.float32)
    O = lax.dot_general(R.astype(jnp.bfloat16), P, (((1,), (1,)), ((), ())),
                        preferred_element_type=jnp.float32)
    li2 = lax.broadcasted_iota(jnp.int32, (N, N), 1)
    si2 = lax.broadcasted_iota(jnp.int32, (N, N), 0)
    vals = O.astype(jnp.int32)
    vals = jnp.where(O > 500.0, -1, vals)
    vals = jnp.where(gly_col & gly_row, vals, -1)
    vals = jnp.where(li2 == si2, 0, vals)
    return vals


def _fw_body(mono_col_ref, mono_row_ref, tb_ref, a2t_ref, out_ref):
    B = a2t_ref.shape[0]
    N = a2t_ref.shape[1]
    T = a2t_ref.shape[2]
    assigns = [_batch_assign(a2t_ref[b], mono_col_ref[b], N, T)
               for b in range(B)]
    Ds = _bfs_closed_pair([tb_ref[b] > 0.0 for b in range(B)],
                          [a[2] for a in assigns], T)
    for b in range(B):
        P, gly_col, _ = assigns[b]
        out_ref[b] = _expand(P, Ds[b], gly_col, mono_row_ref[b] != -1, N)


def kernel(atom_pad_mask, atom_mono_idx, token_bonds, atom_to_token):
    B, N = atom_pad_mask.shape
    T = token_bonds.shape[1]
    tb = jnp.squeeze(token_bonds, -1)
    mono_col = atom_mono_idx.reshape(B, N, 1)
    mono_row = atom_mono_idx.reshape(B, 1, N)
    out = pl.pallas_call(
        _fw_body,
        out_shape=jax.ShapeDtypeStruct((B, N, N), jnp.int32),
    )(mono_col, mono_row, tb, atom_to_token)
    return out
